# Initial kernel scaffold; baseline (speedup 1.0000x reference)
import jax
import jax.numpy as jnp
from jax.experimental import pallas as pl


def kernel(words, tag, word_table, tag_table):
    raise NotImplementedError("write your pallas kernel here")


# speedup vs baseline: 1.8229x; 1.8229x over previous
import functools

import jax
import jax.numpy as jnp
from jax import lax
from jax.experimental import pallas as pl
from jax.experimental.pallas import tpu as pltpu
from jax.experimental.pallas import tpu_sc as plsc

B = 16384
N = 200
D = 64
R = 6.0

_info = plsc.get_sparse_core_info()
NC, NS, L = _info.num_cores, _info.num_subcores, _info.num_lanes
NW = NC * NS
BPW = B // NW
TPG = 128
NTG = BPW // TPG

_mesh = plsc.VectorSubcoreMesh(core_axis_name="c", subcore_axis_name="s")


@functools.partial(
    pl.kernel,
    out_type=[
        jax.ShapeDtypeStruct((B, D), jnp.float32),
        jax.ShapeDtypeStruct((B, D), jnp.float32),
    ],
    mesh=_mesh,
    compiler_params=pltpu.CompilerParams(use_tc_tiling_on_sc=False),
    scratch_types=[
        pltpu.VMEM((2, N // 2), jnp.int32),
        pltpu.VMEM((N, D), jnp.float32),
        pltpu.VMEM((BPW, D), jnp.float32),
        pltpu.VMEM((NTG, TPG), jnp.int32),
        pltpu.VMEM((BPW, D), jnp.float32),
        pltpu.SemaphoreType.DMA,
    ],
)
def _sc_emb(words_hbm, tag_hbm, wtab_hbm, ttab_hbm, se_hbm, tv_hbm,
            widx_v, rows_v, out_v, tidx_v, trows_v, sem):
    wid = lax.axis_index("s") * NC + lax.axis_index("c")
    base = wid * BPW

    pltpu.sync_copy(tag_hbm.at[pl.ds(wid * NTG, NTG)], tidx_v)
    for t in range(NTG):
        pltpu.async_copy(
            ttab_hbm.at[tidx_v.at[t]],
            trows_v.at[pl.ds(t * TPG, TPG)], sem).wait()
    pltpu.sync_copy(trows_v, tv_hbm.at[pl.ds(base, BPW)])

    @pl.loop(0, BPW)
    def _batch(g):
        b = base + g
        pltpu.sync_copy(words_hbm.at[b], widx_v)
        c0 = pltpu.async_copy(
            wtab_hbm.at[widx_v.at[0]], rows_v.at[pl.ds(0, N // 2)], sem)
        c1 = pltpu.async_copy(
            wtab_hbm.at[widx_v.at[1]], rows_v.at[pl.ds(N // 2, N // 2)], sem)
        c0.wait()
        c1.wait()

        zeros = jnp.zeros((L,), jnp.float32)

        @pl.loop(0, N, init_carry=(zeros, zeros, zeros, zeros))
        def _row(n, carry):
            a0, a1, a2, a3 = carry
            a0 = a0 + jnp.exp(rows_v[n, pl.ds(0 * L, L)] * R)
            a1 = a1 + jnp.exp(rows_v[n, pl.ds(1 * L, L)] * R)
            a2 = a2 + jnp.exp(rows_v[n, pl.ds(2 * L, L)] * R)
            a3 = a3 + jnp.exp(rows_v[n, pl.ds(3 * L, L)] * R)
            return a0, a1, a2, a3

        a0, a1, a2, a3 = _row
        out_v[g, pl.ds(0 * L, L)] = a0
        out_v[g, pl.ds(1 * L, L)] = a1
        out_v[g, pl.ds(2 * L, L)] = a2
        out_v[g, pl.ds(3 * L, L)] = a3

    pltpu.sync_copy(out_v, se_hbm.at[pl.ds(base, BPW)])


def _finish_body(tv_ref, se_ref, o_ref):
    pooled = jnp.log(se_ref[:, :]) * (1.0 / R)
    o_ref[:, :] = jnp.concatenate([tv_ref[:, :], pooled], axis=1)


_FIN_BLK = 2048
_finish = pl.pallas_call(
    _finish_body,
    grid=(B // _FIN_BLK,),
    in_specs=[
        pl.BlockSpec((_FIN_BLK, D), lambda i: (i, 0)),
        pl.BlockSpec((_FIN_BLK, D), lambda i: (i, 0)),
    ],
    out_specs=pl.BlockSpec((_FIN_BLK, 2 * D), lambda i: (i, 0)),
    out_shape=jax.ShapeDtypeStruct((B, 2 * D), jnp.float32),
)


@jax.jit
def kernel(words, tag, word_table, tag_table):
    words3 = words.reshape(B, 2, N // 2)
    tag2 = tag.reshape(B // TPG, TPG)
    sum_exp, tag_vecs = _sc_emb(words3, tag2, word_table, tag_table)
    return _finish(tag_vecs, sum_exp)


# speedup vs baseline: 2.9739x; 1.6314x over previous
import functools

import jax
import jax.numpy as jnp
from jax import lax
from jax.experimental import pallas as pl
from jax.experimental.pallas import tpu as pltpu
from jax.experimental.pallas import tpu_sc as plsc

B = 16384
N = 200
D = 64
R = 6.0

_info = plsc.get_sparse_core_info()
NC, NS, L = _info.num_cores, _info.num_subcores, _info.num_lanes
NW = NC * NS
BPW = B // NW
TPG = 128
NTG = BPW // TPG
IC = 64
NCH = BPW // IC
NBUF = 4
H = N // 2

_mesh = plsc.VectorSubcoreMesh(core_axis_name="c", subcore_axis_name="s")


@functools.partial(
    pl.kernel,
    out_type=[
        jax.ShapeDtypeStruct((B, D), jnp.float32),
        jax.ShapeDtypeStruct((B, D), jnp.float32),
    ],
    mesh=_mesh,
    compiler_params=pltpu.CompilerParams(use_tc_tiling_on_sc=False),
    scratch_types=[
        pltpu.VMEM((IC, 2, H), jnp.int32),
        pltpu.VMEM((NBUF, N, D), jnp.float32),
        pltpu.VMEM((IC, D), jnp.float32),
        pltpu.VMEM((NTG, TPG), jnp.int32),
        pltpu.VMEM((TPG, D), jnp.float32),
        pltpu.SemaphoreType.DMA,
        pltpu.SemaphoreType.DMA,
        pltpu.SemaphoreType.DMA,
        pltpu.SemaphoreType.DMA,
        pltpu.SemaphoreType.DMA,
    ],
)
def _sc_emb(words_hbm, tag_hbm, wtab_hbm, ttab_hbm, se_hbm, tv_hbm,
            ichunk, rows_v, out_v, tidx_v, trows_v, s0, s1, s2, s3, tsem):
    sems = (s0, s1, s2, s3)
    wid = lax.axis_index("s") * NC + lax.axis_index("c")
    base = wid * BPW

    pltpu.sync_copy(tag_hbm.at[pl.ds(wid * NTG, NTG)], tidx_v)
    for t in range(NTG):
        pltpu.async_copy(ttab_hbm.at[tidx_v.at[t]], trows_v, tsem).wait()
        pltpu.sync_copy(trows_v, tv_hbm.at[pl.ds(base + t * TPG, TPG)])

    def fire(s, local):
        pltpu.async_copy(wtab_hbm.at[ichunk.at[local, 0]],
                         rows_v.at[s, pl.ds(0, H)], sems[s])
        pltpu.async_copy(wtab_hbm.at[ichunk.at[local, 1]],
                         rows_v.at[s, pl.ds(H, H)], sems[s])

    @pl.loop(0, NCH)
    def _chunk(c):
        cb = base + c * IC
        pltpu.sync_copy(words_hbm.at[pl.ds(cb, IC)], ichunk)
        for s in range(NBUF):
            fire(s, s)

        @pl.loop(0, IC, step=NBUF)
        def _group(l):
            for s in range(NBUF):
                local = l + s
                pltpu.make_async_copy(
                    wtab_hbm.at[pl.ds(0, N)], rows_v.at[s], sems[s]).wait()

                zeros = jnp.zeros((L,), jnp.float32)

                @pl.loop(0, N, init_carry=(zeros, zeros, zeros, zeros))
                def _row(n, carry):
                    a0, a1, a2, a3 = carry
                    a0 = a0 + jnp.exp(rows_v[s, n, pl.ds(0 * L, L)] * R)
                    a1 = a1 + jnp.exp(rows_v[s, n, pl.ds(1 * L, L)] * R)
                    a2 = a2 + jnp.exp(rows_v[s, n, pl.ds(2 * L, L)] * R)
                    a3 = a3 + jnp.exp(rows_v[s, n, pl.ds(3 * L, L)] * R)
                    return a0, a1, a2, a3

                a0, a1, a2, a3 = _row
                out_v[local, pl.ds(0 * L, L)] = a0
                out_v[local, pl.ds(1 * L, L)] = a1
                out_v[local, pl.ds(2 * L, L)] = a2
                out_v[local, pl.ds(3 * L, L)] = a3

                @pl.when(local + NBUF < IC)
                def _():
                    fire(s, local + NBUF)

        pltpu.sync_copy(out_v, se_hbm.at[pl.ds(cb, IC)])


def _finish_body(tv_ref, se_ref, o_ref):
    pooled = jnp.log(se_ref[:, :]) * (1.0 / R)
    o_ref[:, :] = jnp.concatenate([tv_ref[:, :], pooled], axis=1)


_FIN_BLK = 2048
_finish = pl.pallas_call(
    _finish_body,
    grid=(B // _FIN_BLK,),
    in_specs=[
        pl.BlockSpec((_FIN_BLK, D), lambda i: (i, 0)),
        pl.BlockSpec((_FIN_BLK, D), lambda i: (i, 0)),
    ],
    out_specs=pl.BlockSpec((_FIN_BLK, 2 * D), lambda i: (i, 0)),
    out_shape=jax.ShapeDtypeStruct((B, 2 * D), jnp.float32),
)


@jax.jit
def kernel(words, tag, word_table, tag_table):
    words3 = words.reshape(B, 2, H)
    tag2 = tag.reshape(B // TPG, TPG)
    sum_exp, tag_vecs = _sc_emb(words3, tag2, word_table, tag_table)
    return _finish(tag_vecs, sum_exp)


# speedup vs baseline: 3.0384x; 1.0217x over previous
import functools

import jax
import jax.numpy as jnp
from jax import lax
from jax.experimental import pallas as pl
from jax.experimental.pallas import tpu as pltpu
from jax.experimental.pallas import tpu_sc as plsc

B = 16384
N = 200
D = 64
R = 6.0

_info = plsc.get_sparse_core_info()
NC, NS, L = _info.num_cores, _info.num_subcores, _info.num_lanes
NW = NC * NS
BPW = B // NW
TPG = 128
NTG = BPW // TPG
IC = 64
NCH = BPW // IC
NBUF = 4
H1 = 120
H2 = N - H1

_C = (1.0000006974281586, -0.5000073548516979, 0.3331793391436614,
      -0.2492950419943796, 0.2045542018978282, -0.1845583495672427,
      0.11784427706676123)
_SQRT2 = 1.4142135623730951
_LN2 = 0.6931471805599453

_mesh = plsc.VectorSubcoreMesh(core_axis_name="c", subcore_axis_name="s")


def _log_over_r(a):
    bits = plsc.bitcast(a, jnp.int32)
    e = lax.shift_right_logical(bits, 23) - 127
    m = plsc.bitcast((bits & 0x007FFFFF) | 0x3F800000, jnp.float32)
    adj = m >= _SQRT2
    m = jnp.where(adj, m * 0.5, m)
    e = (e + adj.astype(jnp.int32)).astype(jnp.float32)
    t = m - 1.0
    p = jnp.float32(_C[6])
    for k in range(5, -1, -1):
        p = p * t + _C[k]
    return (e * _LN2 + t * p) * (1.0 / R)


@functools.partial(
    pl.kernel,
    out_type=jax.ShapeDtypeStruct((B, 2 * D), jnp.float32),
    mesh=_mesh,
    compiler_params=pltpu.CompilerParams(
        use_tc_tiling_on_sc=False, needs_layout_passes=False),
    scratch_types=[
        pltpu.VMEM((IC, H1), jnp.int32),
        pltpu.VMEM((IC, H2), jnp.int32),
        pltpu.VMEM((NBUF, N, D), jnp.float32),
        pltpu.VMEM((IC, D), jnp.float32),
        pltpu.VMEM((TPG,), jnp.int32),
        pltpu.VMEM((TPG, D), jnp.float32),
        pltpu.SemaphoreType.DMA,
        pltpu.SemaphoreType.DMA,
        pltpu.SemaphoreType.DMA,
        pltpu.SemaphoreType.DMA,
        pltpu.SemaphoreType.DMA,
    ],
)
def _sc_emb(words_hbm, tag_hbm, wtab_hbm, ttab_hbm, out_hbm,
            ich0, ich1, rows_v, out_v, tidx_v, trows_v, s0, s1, s2, s3, tsem):
    sems = (s0, s1, s2, s3)
    wid = lax.axis_index("s") * NC + lax.axis_index("c")
    base = wid * BPW

    for t in range(NTG):
        pltpu.sync_copy(tag_hbm.at[pl.ds(base + t * TPG, TPG)], tidx_v)
        pltpu.async_copy(ttab_hbm.at[tidx_v], trows_v, tsem).wait()
        pltpu.sync_copy(
            trows_v, out_hbm.at[pl.ds(base + t * TPG, TPG), pl.ds(0, D)])

    def fire(s, local):
        pltpu.async_copy(wtab_hbm.at[ich0.at[local]],
                         rows_v.at[s, pl.ds(0, H1)], sems[s])
        pltpu.async_copy(wtab_hbm.at[ich1.at[local]],
                         rows_v.at[s, pl.ds(H1, H2)], sems[s])

    @pl.loop(0, NCH)
    def _chunk(c):
        cb = base + c * IC
        pltpu.sync_copy(words_hbm.at[pl.ds(cb, IC), pl.ds(0, H1)], ich0)
        pltpu.sync_copy(words_hbm.at[pl.ds(cb, IC), pl.ds(H1, H2)], ich1)
        for s in range(NBUF):
            fire(s, s)

        @pl.loop(0, IC, step=NBUF)
        def _group(l):
            for s in range(NBUF):
                local = l + s
                pltpu.make_async_copy(
                    wtab_hbm.at[pl.ds(0, N)], rows_v.at[s], sems[s]).wait()

                zeros = jnp.zeros((L,), jnp.float32)

                @pl.loop(0, N, init_carry=(zeros, zeros, zeros, zeros))
                def _row(n, carry):
                    a0, a1, a2, a3 = carry
                    a0 = a0 + jnp.exp(rows_v[s, n, pl.ds(0 * L, L)] * R)
                    a1 = a1 + jnp.exp(rows_v[s, n, pl.ds(1 * L, L)] * R)
                    a2 = a2 + jnp.exp(rows_v[s, n, pl.ds(2 * L, L)] * R)
                    a3 = a3 + jnp.exp(rows_v[s, n, pl.ds(3 * L, L)] * R)
                    return a0, a1, a2, a3

                a0, a1, a2, a3 = _row
                out_v[local, pl.ds(0 * L, L)] = _log_over_r(a0)
                out_v[local, pl.ds(1 * L, L)] = _log_over_r(a1)
                out_v[local, pl.ds(2 * L, L)] = _log_over_r(a2)
                out_v[local, pl.ds(3 * L, L)] = _log_over_r(a3)

                @pl.when(local + NBUF < IC)
                def _():
                    fire(s, local + NBUF)

        pltpu.sync_copy(out_v, out_hbm.at[pl.ds(cb, IC), pl.ds(D, D)])


@jax.jit
def kernel(words, tag, word_table, tag_table):
    return _sc_emb(words, tag, word_table, tag_table)


# speedup vs baseline: 3.0605x; 1.0073x over previous
import functools

import jax
import jax.numpy as jnp
from jax import lax
from jax.experimental import pallas as pl
from jax.experimental.pallas import tpu as pltpu
from jax.experimental.pallas import tpu_sc as plsc

B = 16384
N = 200
D = 64
R = 6.0

_info = plsc.get_sparse_core_info()
NC, NS, L = _info.num_cores, _info.num_subcores, _info.num_lanes
NW = NC * NS
BPW = B // NW
TPG = 128
NTG = BPW // TPG
IC = 64
NCH = BPW // IC
NBUF = 4
H1 = 120
H2 = N - H1

_C = (1.0000006974281586, -0.5000073548516979, 0.3331793391436614,
      -0.2492950419943796, 0.2045542018978282, -0.1845583495672427,
      0.11784427706676123)
_SQRT2 = 1.4142135623730951
_LN2 = 0.6931471805599453

_mesh = plsc.VectorSubcoreMesh(core_axis_name="c", subcore_axis_name="s")


def _log_over_r(a):
    bits = plsc.bitcast(a, jnp.int32)
    e = lax.shift_right_logical(bits, 23) - 127
    m = plsc.bitcast((bits & 0x007FFFFF) | 0x3F800000, jnp.float32)
    adj = m >= _SQRT2
    m = jnp.where(adj, m * 0.5, m)
    e = (e + adj.astype(jnp.int32)).astype(jnp.float32)
    t = m - 1.0
    p = jnp.float32(_C[6])
    for k in range(5, -1, -1):
        p = p * t + _C[k]
    return (e * _LN2 + t * p) * (1.0 / R)


@functools.partial(
    pl.kernel,
    out_type=jax.ShapeDtypeStruct((B, 2 * D), jnp.float32),
    mesh=_mesh,
    compiler_params=pltpu.CompilerParams(
        use_tc_tiling_on_sc=False, needs_layout_passes=False),
    scratch_types=[
        pltpu.VMEM((IC * N,), jnp.int32),
        pltpu.VMEM((NBUF, N, D), jnp.float32),
        pltpu.VMEM((IC, D), jnp.float32),
        pltpu.VMEM((TPG,), jnp.int32),
        pltpu.VMEM((TPG, D), jnp.float32),
        pltpu.SemaphoreType.DMA,
        pltpu.SemaphoreType.DMA,
        pltpu.SemaphoreType.DMA,
        pltpu.SemaphoreType.DMA,
        pltpu.SemaphoreType.DMA,
    ],
)
def _sc_emb(words_hbm, tag_hbm, wtab_hbm, ttab_hbm, out_hbm,
            ichunk, rows_v, out_v, tidx_v, trows_v, s0, s1, s2, s3, tsem):
    sems = (s0, s1, s2, s3)
    wid = lax.axis_index("s") * NC + lax.axis_index("c")
    base = wid * BPW

    for t in range(NTG):
        pltpu.sync_copy(tag_hbm.at[pl.ds(base + t * TPG, TPG)], tidx_v)
        pltpu.async_copy(ttab_hbm.at[tidx_v], trows_v, tsem).wait()
        pltpu.sync_copy(
            trows_v, out_hbm.at[pl.ds(base + t * TPG, TPG), pl.ds(0, D)])

    def fire(s, local):
        off = pl.multiple_of(local * N, 8)
        pltpu.async_copy(wtab_hbm.at[ichunk.at[pl.ds(off, H1)]],
                         rows_v.at[s, pl.ds(0, H1)], sems[s])
        pltpu.async_copy(wtab_hbm.at[ichunk.at[pl.ds(off + H1, H2)]],
                         rows_v.at[s, pl.ds(H1, H2)], sems[s])

    @pl.loop(0, NCH)
    def _chunk(c):
        cb = base + c * IC
        pltpu.sync_copy(words_hbm.at[pl.ds(cb * N, IC * N)], ichunk)
        for s in range(NBUF):
            fire(s, s)

        @pl.loop(0, IC, step=NBUF)
        def _group(l):
            for s in range(NBUF):
                local = l + s
                pltpu.make_async_copy(
                    wtab_hbm.at[pl.ds(0, N)], rows_v.at[s], sems[s]).wait()

                zeros = jnp.zeros((L,), jnp.float32)

                @pl.loop(0, N, init_carry=(zeros, zeros, zeros, zeros))
                def _row(n, carry):
                    a0, a1, a2, a3 = carry
                    a0 = a0 + jnp.exp(rows_v[s, n, pl.ds(0 * L, L)] * R)
                    a1 = a1 + jnp.exp(rows_v[s, n, pl.ds(1 * L, L)] * R)
                    a2 = a2 + jnp.exp(rows_v[s, n, pl.ds(2 * L, L)] * R)
                    a3 = a3 + jnp.exp(rows_v[s, n, pl.ds(3 * L, L)] * R)
                    return a0, a1, a2, a3

                a0, a1, a2, a3 = _row
                out_v[local, pl.ds(0 * L, L)] = _log_over_r(a0)
                out_v[local, pl.ds(1 * L, L)] = _log_over_r(a1)
                out_v[local, pl.ds(2 * L, L)] = _log_over_r(a2)
                out_v[local, pl.ds(3 * L, L)] = _log_over_r(a3)

                @pl.when(local + NBUF < IC)
                def _():
                    fire(s, local + NBUF)

        pltpu.sync_copy(out_v, out_hbm.at[pl.ds(cb, IC), pl.ds(D, D)])


@jax.jit
def kernel(words, tag, word_table, tag_table):
    return _sc_emb(words.reshape(-1), tag, word_table, tag_table)
